# baseline (speedup 1.0000x reference)
import functools

import jax
import jax.numpy as jnp
from jax import lax
from jax.experimental import pallas as pl
from jax.experimental.pallas import tpu as pltpu
from jax.experimental.pallas import tpu_sc as plsc

E = 128
EJ = E // 16
NC = 2
NS = 16
NW = NC * NS
CH = 320
GRP = CH // 16


def _rsqrt_newton(x):
    i = plsc.bitcast(x, jnp.int32)
    i = jnp.int32(0x5F3759DF) - lax.shift_right_logical(i, 1)
    y = plsc.bitcast(i, jnp.float32)
    half = x * 0.5
    for _ in range(3):
        y = y * (1.5 - half * y * y)
    return y


def _fused_body(l_seq, seq_hbm, table_hbm, pe_hbm, gamma_hbm, beta_hbm,
                out_hbm, idx_v, rows_a, rows_b, pe_v, gb_v,
                sem_ga, sem_gb, sem_oa, sem_ob):
    t_total = seq_hbm.shape[0]
    tpw = t_total // NW
    nch = tpw // CH
    wid = lax.axis_index("s") * NC + lax.axis_index("c")
    base = wid * tpw

    pltpu.sync_copy(seq_hbm.at[pl.ds(base, tpw)], idx_v)
    pltpu.sync_copy(pe_hbm, pe_v)
    pltpu.sync_copy(gamma_hbm, gb_v.at[0])
    pltpu.sync_copy(beta_hbm, gb_v.at[1])

    bufs = (rows_a, rows_b)
    g_sems = (sem_ga, sem_gb)
    o_sems = (sem_oa, sem_ob)

    def start_gather(c, b):
        pltpu.async_copy(table_hbm.at[idx_v.at[pl.ds(c * CH, CH)]],
                         bufs[b], g_sems[b])

    def wait_gather(c, b):
        pltpu.make_async_copy(table_hbm.at[idx_v.at[pl.ds(c * CH, CH)]],
                              bufs[b], g_sems[b]).wait()

    def start_out(c, b):
        pltpu.async_copy(bufs[b], out_hbm.at[pl.ds(base + c * CH, CH)],
                         o_sems[b])

    def wait_out(c, b):
        pltpu.make_async_copy(bufs[b], out_hbm.at[pl.ds(base + c * CH, CH)],
                              o_sems[b]).wait()

    def compute_chunk(c, b):
        rows = bufs[b]
        gs = [gb_v[0, pl.ds(j * 16, 16)] for j in range(EJ)]
        bs = [gb_v[1, pl.ds(j * 16, 16)] for j in range(EJ)]

        lane = lax.iota(jnp.int32, 16)
        zero16 = jnp.zeros((16,), jnp.float32)

        def _tree_sum(vs):
            while len(vs) > 1:
                vs = [vs[i] + vs[i + 1] for i in range(0, len(vs) - 1, 2)] + (
                    [vs[-1]] if len(vs) % 2 else [])
            return vs[0]

        def group_body(g, _):
            g16 = g * 16
            tid16 = idx_v[pl.ds(c * CH + g16, 16)]
            cnt0 = plsc.all_reduce_population_count(tid16 == 0)

            @pl.when(cnt0[0] > 0)
            def _():
                for k in range(16):
                    @pl.when(tid16[k] == 0)
                    def _():
                        for j in range(EJ):
                            rows[g16 + k, pl.ds(j * 16, 16)] = zero16

            pos0 = lax.rem(base + c * CH + g16, l_seq)
            for sub in range(4):
                sums = zero16
                sqs = zero16
                es = []
                for k4 in range(4):
                    k = sub * 4 + k4
                    t = g16 + k
                    pos = pos0 + k
                    pos = jnp.where(pos >= l_seq, pos - l_seq, pos)
                    e = [rows[t, pl.ds(j * 16, 16)] + pe_v[pos, pl.ds(j * 16, 16)]
                         for j in range(EJ)]
                    s = _tree_sum(e)
                    q = _tree_sum([x * x for x in e])
                    sk = jnp.full((16,), jnp.sum(s), jnp.float32)
                    qk = jnp.full((16,), jnp.sum(q), jnp.float32)
                    sums = jnp.where(lane == k4, sk, sums)
                    sqs = jnp.where(lane == k4, qk, sqs)
                    es.append(e)
                mean4 = sums * (1.0 / E)
                var4 = sqs * (1.0 / E) - mean4 * mean4
                rstd4 = _rsqrt_newton(var4 + 1e-12)
                for k4 in range(4):
                    t = g16 + sub * 4 + k4
                    mk = jnp.full((16,), mean4[k4], jnp.float32)
                    rk = jnp.full((16,), rstd4[k4], jnp.float32)
                    for j in range(EJ):
                        rows[t, pl.ds(j * 16, 16)] = (es[k4][j] - mk) * rk * gs[j] + bs[j]
            return 0

        lax.fori_loop(0, GRP, group_body, 0)

    start_gather(0, 0)

    def chunk_pair(i, _):
        c2 = i * 2
        for b in (0, 1):
            c = c2 + b
            wait_gather(c, b)
            nb = 1 - b

            @pl.when(c >= 1)
            def _():
                wait_out(c - 1, nb)

            @pl.when(c + 1 < nch)
            def _():
                start_gather(c + 1, nb)

            compute_chunk(c, b)
            start_out(c, b)
        return 0

    lax.fori_loop(0, nch // 2, chunk_pair, 0)
    wait_out(nch - 1, (nch - 1) % 2)


def kernel(sequence, table, gamma, beta, pe):
    b, l = sequence.shape
    t_total = b * l
    seq_flat = sequence.reshape(-1).astype(jnp.int32)
    pe_l = pe[:l]
    mesh = plsc.VectorSubcoreMesh(core_axis_name="c", subcore_axis_name="s")
    fn = pl.kernel(
        functools.partial(_fused_body, l),
        out_type=jax.ShapeDtypeStruct((t_total, E), jnp.float32),
        mesh=mesh,
        compiler_params=pltpu.CompilerParams(needs_layout_passes=False),
        scratch_types=[
            pltpu.VMEM((t_total // NW,), jnp.int32),
            pltpu.VMEM((CH, E), jnp.float32),
            pltpu.VMEM((CH, E), jnp.float32),
            pltpu.VMEM((l, E), jnp.float32),
            pltpu.VMEM((2, E), jnp.float32),
            pltpu.SemaphoreType.DMA,
            pltpu.SemaphoreType.DMA,
            pltpu.SemaphoreType.DMA,
            pltpu.SemaphoreType.DMA,
        ],
    )
    out = fn(seq_flat, table, pe_l, gamma, beta)
    return out.reshape(b, l, E)


# speedup vs baseline: 1.1479x; 1.1479x over previous
import functools

import jax
import jax.numpy as jnp
from jax import lax
from jax.experimental import pallas as pl
from jax.experimental.pallas import tpu as pltpu
from jax.experimental.pallas import tpu_sc as plsc

E = 128
EJ = E // 16
NC = 2
NS = 16
NW = NC * NS
CH = 320
GRP = CH // 16


def _rsqrt_newton(x):
    i = plsc.bitcast(x, jnp.int32)
    i = jnp.int32(0x5F3759DF) - lax.shift_right_logical(i, 1)
    y = plsc.bitcast(i, jnp.float32)
    half = x * 0.5
    for _ in range(3):
        y = y * (1.5 - half * y * y)
    return y


def _fused_body(l_seq, seq_hbm, table_hbm, pe_hbm, gamma_hbm, beta_hbm,
                out_hbm, idx_v, rows_a, rows_b, pe_v, gb_v,
                sem_ga, sem_gb, sem_oa, sem_ob):
    t_total = seq_hbm.shape[0]
    tpw = t_total // NW
    nch = tpw // CH
    wid = lax.axis_index("s") * NC + lax.axis_index("c")
    base = wid * tpw

    pltpu.sync_copy(seq_hbm.at[pl.ds(base, tpw)], idx_v)
    pltpu.sync_copy(pe_hbm, pe_v)
    pltpu.sync_copy(gamma_hbm, gb_v.at[0])
    pltpu.sync_copy(beta_hbm, gb_v.at[1])

    bufs = (rows_a, rows_b)
    g_sems = (sem_ga, sem_gb)
    o_sems = (sem_oa, sem_ob)

    def start_gather(c, b):
        pltpu.async_copy(table_hbm.at[idx_v.at[pl.ds(c * CH, CH)]],
                         bufs[b], g_sems[b])

    def wait_gather(c, b):
        pltpu.make_async_copy(table_hbm.at[idx_v.at[pl.ds(c * CH, CH)]],
                              bufs[b], g_sems[b]).wait()

    def start_out(c, b):
        pltpu.async_copy(bufs[b], out_hbm.at[pl.ds(base + c * CH, CH)],
                         o_sems[b])

    def wait_out(c, b):
        pltpu.make_async_copy(bufs[b], out_hbm.at[pl.ds(base + c * CH, CH)],
                              o_sems[b]).wait()

    def compute_chunk(c, b):
        rows = bufs[b]
        gs = [gb_v[0, pl.ds(j * 16, 16)] for j in range(EJ)]
        bs = [gb_v[1, pl.ds(j * 16, 16)] for j in range(EJ)]

        lane = lax.iota(jnp.int32, 16)
        zero16 = jnp.zeros((16,), jnp.float32)

        def _tree_sum(vs):
            while len(vs) > 1:
                vs = [vs[i] + vs[i + 1] for i in range(0, len(vs) - 1, 2)] + (
                    [vs[-1]] if len(vs) % 2 else [])
            return vs[0]

        def group_body(g, _):
            g16 = g * 16
            tid16 = idx_v[pl.ds(c * CH + g16, 16)]
            cnt0 = plsc.all_reduce_population_count(tid16 == 0)

            @pl.when(cnt0[0] > 0)
            def _():
                for k in range(16):
                    @pl.when(tid16[k] == 0)
                    def _():
                        for j in range(EJ):
                            rows[g16 + k, pl.ds(j * 16, 16)] = zero16

            pos0 = lax.rem(base + c * CH + g16, l_seq)
            sums = zero16
            sqs = zero16
            for k in range(16):
                t = g16 + k
                pos = pos0 + k
                pos = jnp.where(pos >= l_seq, pos - l_seq, pos)
                e = [rows[t, pl.ds(j * 16, 16)] + pe_v[pos, pl.ds(j * 16, 16)]
                     for j in range(EJ)]
                s = _tree_sum(e)
                q = _tree_sum([x * x for x in e])
                for j in range(EJ):
                    rows[t, pl.ds(j * 16, 16)] = e[j]
                sk = jnp.full((16,), jnp.sum(s), jnp.float32)
                qk = jnp.full((16,), jnp.sum(q), jnp.float32)
                sums = jnp.where(lane == k, sk, sums)
                sqs = jnp.where(lane == k, qk, sqs)
            mean16 = sums * (1.0 / E)
            var16 = sqs * (1.0 / E) - mean16 * mean16
            rstd16 = _rsqrt_newton(var16 + 1e-12)
            for k in range(16):
                t = g16 + k
                mk = jnp.full((16,), mean16[k], jnp.float32)
                rk = jnp.full((16,), rstd16[k], jnp.float32)
                for j in range(EJ):
                    ej = rows[t, pl.ds(j * 16, 16)]
                    rows[t, pl.ds(j * 16, 16)] = (ej - mk) * rk * gs[j] + bs[j]
            return 0

        lax.fori_loop(0, GRP, group_body, 0)

    start_gather(0, 0)

    def chunk_pair(i, _):
        c2 = i * 2
        for b in (0, 1):
            c = c2 + b
            wait_gather(c, b)
            nb = 1 - b

            @pl.when(c >= 1)
            def _():
                wait_out(c - 1, nb)

            @pl.when(c + 1 < nch)
            def _():
                start_gather(c + 1, nb)

            compute_chunk(c, b)
            start_out(c, b)
        return 0

    lax.fori_loop(0, nch // 2, chunk_pair, 0)
    wait_out(nch - 1, (nch - 1) % 2)


def kernel(sequence, table, gamma, beta, pe):
    b, l = sequence.shape
    t_total = b * l
    seq_flat = sequence.reshape(-1).astype(jnp.int32)
    pe_l = pe[:l]
    mesh = plsc.VectorSubcoreMesh(core_axis_name="c", subcore_axis_name="s")
    fn = pl.kernel(
        functools.partial(_fused_body, l),
        out_type=jax.ShapeDtypeStruct((t_total, E), jnp.float32),
        mesh=mesh,
        compiler_params=pltpu.CompilerParams(needs_layout_passes=False),
        scratch_types=[
            pltpu.VMEM((t_total // NW,), jnp.int32),
            pltpu.VMEM((CH, E), jnp.float32),
            pltpu.VMEM((CH, E), jnp.float32),
            pltpu.VMEM((l, E), jnp.float32),
            pltpu.VMEM((2, E), jnp.float32),
            pltpu.SemaphoreType.DMA,
            pltpu.SemaphoreType.DMA,
            pltpu.SemaphoreType.DMA,
            pltpu.SemaphoreType.DMA,
        ],
    )
    out = fn(seq_flat, table, pe_l, gamma, beta)
    return out.reshape(b, l, E)
